# Initial kernel scaffold; baseline (speedup 1.0000x reference)
import jax
import jax.numpy as jnp
from jax.experimental import pallas as pl


def kernel(x, c1_w, c2_w, fc1_w, fc1_b, fc2_w, fc2_b):
    raise NotImplementedError("write your pallas kernel here")


# speedup vs baseline: 2.5965x; 2.5965x over previous
import jax
import jax.numpy as jnp
from jax.experimental import pallas as pl
from jax.experimental.pallas import tpu as pltpu

GRID_SIZE = 5
SPLINE_ORDER = 3
N_COEF = GRID_SIZE + SPLINE_ORDER
N_KNOTS = GRID_SIZE + 2 * SPLINE_ORDER + 1
H_STEP = 2.0 / GRID_SIZE
KNOTS = tuple(-1.0 + H_STEP * (i - SPLINE_ORDER) for i in range(N_KNOTS))
N_CONVS = 5
N_GROUPS = 1 + N_COEF
KK = 9

TP = 4096
HALO = 128
HIDDEN = 256
N_CLASSES = 3
FC2_PAD = 128
FC1_IN = 625
FC1_IN_PAD = 640

OFFS28 = tuple(di * 28 + dj for di in range(3) for dj in range(3))
OFFS13 = tuple(di * 13 + dj for di in range(3) for dj in range(3))


def _featurize(x):
    g = KNOTS
    c = [(x >= g[j]).astype(jnp.float32) for j in range(N_KNOTS)]
    bases = [c[j] - c[j + 1] for j in range(N_KNOTS - 1)]
    for k in range(1, SPLINE_ORDER + 1):
        inv_k = 1.0 / (k * H_STEP)
        xk = x * inv_k
        new_bases = []
        t_prev = xk - g[0] * inv_k
        for j in range(len(bases) - 1):
            t_next = xk - g[j + 1] * inv_k
            new_bases.append(t_prev * bases[j] + (1.0 - t_next) * bases[j + 1])
            t_prev = t_next
        bases = new_bases
    silu = x * jax.nn.sigmoid(x)
    return jnp.concatenate([silu] + bases, axis=0)


def _make_conv_body(offs):
    def body(x_ref, h_ref, w_ref, o_ref):
        xf = jnp.concatenate([x_ref[...], h_ref[...]], axis=1)
        feat = _featurize(xf)
        a = jnp.concatenate([feat[:, o:o + TP] for o in offs], axis=0)
        o_ref[...] = jnp.maximum(
            jnp.dot(w_ref[...], a, preferred_element_type=jnp.float32), 0.0)
    return body


def _kan_conv_relu(xflat, w81, offs):
    P = xflat.shape[1]
    n = pl.cdiv(P, TP)
    ppad = n * TP
    xbig = jnp.pad(xflat, ((0, 0), (0, ppad + TP - P)))
    out = pl.pallas_call(
        _make_conv_body(offs),
        out_shape=jax.ShapeDtypeStruct((N_CONVS, ppad), jnp.float32),
        grid=(n,),
        in_specs=[
            pl.BlockSpec((1, TP), lambda i: (0, i)),
            pl.BlockSpec((1, HALO), lambda i: (0, (i + 1) * (TP // HALO))),
            pl.BlockSpec((N_CONVS, N_GROUPS * KK), lambda i: (0, 0)),
        ],
        out_specs=pl.BlockSpec((N_CONVS, TP), lambda i: (0, i)),
        compiler_params=pltpu.CompilerParams(dimension_semantics=("parallel",)),
    )(xbig, xbig, w81)
    return out[:, :P]


def _pool2(x):
    *lead, h, w = x.shape
    x = x[..., : h // 2 * 2, : w // 2 * 2]
    x = x.reshape(*lead, h // 2, 2, w // 2, 2)
    return x.max(axis=(-3, -1))


def _mlp_body(x_ref, w1_ref, b1_ref, w2_ref, b2_ref, o_ref):
    h = jnp.dot(x_ref[...], w1_ref[...], preferred_element_type=jnp.float32)
    h = jnp.maximum(h + b1_ref[...], 0.0)
    z = jnp.dot(h, w2_ref[...], preferred_element_type=jnp.float32) + b2_ref[...]
    col = jax.lax.broadcasted_iota(jnp.int32, z.shape, 1)
    valid = col < N_CLASSES
    m = jnp.max(jnp.where(valid, z, -jnp.inf), axis=-1, keepdims=True)
    e = jnp.where(valid, jnp.exp(z - m), 0.0)
    lse = jnp.log(jnp.sum(e, axis=-1, keepdims=True)) + m
    o_ref[...] = jnp.where(valid, z - lse, 0.0)


def kernel(x, c1_w, c2_w, fc1_w, fc1_b, fc2_w, fc2_b):
    B = x.shape[0]
    w81_1 = jnp.transpose(c1_w, (1, 2, 0)).reshape(N_CONVS, N_GROUPS * KK)
    w81_2 = jnp.transpose(c2_w, (1, 2, 0)).reshape(N_CONVS, N_GROUPS * KK)

    o1 = _kan_conv_relu(x.reshape(1, B * 28 * 28), w81_1, OFFS28)
    o1 = o1.reshape(N_CONVS, B, 28, 28)[:, :, :26, :26]
    p1 = _pool2(o1)

    o2 = _kan_conv_relu(p1.reshape(1, N_CONVS * B * 13 * 13), w81_2, OFFS13)
    o2 = o2.reshape(N_CONVS, N_CONVS, B, 13, 13)[..., :11, :11]
    p2 = _pool2(o2)

    feats = p2.reshape(25, B, 25).transpose(1, 0, 2).reshape(B, FC1_IN)
    feats = jnp.pad(feats, ((0, 0), (0, FC1_IN_PAD - FC1_IN)))
    w1 = jnp.pad(fc1_w.reshape(FC1_IN, HIDDEN),
                 ((0, FC1_IN_PAD - FC1_IN), (0, 0)))

    bt = 256
    bpad = pl.cdiv(B, bt) * bt
    feats = jnp.pad(feats, ((0, bpad - B), (0, 0)))
    out = pl.pallas_call(
        _mlp_body,
        out_shape=jax.ShapeDtypeStruct((bpad, FC2_PAD), jnp.float32),
        grid=(bpad // bt,),
        in_specs=[
            pl.BlockSpec((bt, FC1_IN_PAD), lambda i: (i, 0)),
            pl.BlockSpec((FC1_IN_PAD, HIDDEN), lambda i: (0, 0)),
            pl.BlockSpec((1, HIDDEN), lambda i: (0, 0)),
            pl.BlockSpec((HIDDEN, FC2_PAD), lambda i: (0, 0)),
            pl.BlockSpec((1, FC2_PAD), lambda i: (0, 0)),
        ],
        out_specs=pl.BlockSpec((bt, FC2_PAD), lambda i: (i, 0)),
        compiler_params=pltpu.CompilerParams(dimension_semantics=("parallel",)),
    )(feats, w1, fc1_b, fc2_w, fc2_b)
    return out[:B, :N_CLASSES]


# speedup vs baseline: 3.0633x; 1.1798x over previous
import jax
import jax.numpy as jnp
from jax.experimental import pallas as pl
from jax.experimental.pallas import tpu as pltpu

GRID_SIZE = 5
SPLINE_ORDER = 3
N_COEF = GRID_SIZE + SPLINE_ORDER
N_KNOTS = GRID_SIZE + 2 * SPLINE_ORDER + 1
H_STEP = 2.0 / GRID_SIZE
KNOTS = tuple(-1.0 + H_STEP * (i - SPLINE_ORDER) for i in range(N_KNOTS))
N_CONVS = 5
N_GROUPS = 1 + N_COEF
KK = 9

TP = 4096
HALO = 128
HIDDEN = 256
N_CLASSES = 3
FC2_PAD = 128
FC1_IN = 625
FC1_IN_PAD = 640

OFFS28 = tuple(di * 28 + dj for di in range(3) for dj in range(3))
OFFS13 = tuple(di * 13 + dj for di in range(3) for dj in range(3))


def _featurize(x):
    g = KNOTS
    c = [(x >= g[j]).astype(jnp.float32) for j in range(N_KNOTS)]
    bases = [c[j] - c[j + 1] for j in range(N_KNOTS - 1)]
    for k in range(1, SPLINE_ORDER + 1):
        inv_k = 1.0 / (k * H_STEP)
        xk = x * inv_k
        new_bases = []
        t_prev = xk - g[0] * inv_k
        for j in range(len(bases) - 1):
            t_next = xk - g[j + 1] * inv_k
            new_bases.append(t_prev * bases[j] + (1.0 - t_next) * bases[j + 1])
            t_prev = t_next
        bases = new_bases
    silu = x * jax.nn.sigmoid(x)
    return [silu] + bases


def _make_conv_body(offs):
    def body(x_ref, h_ref, w_ref, o_ref):
        xf = jnp.concatenate([x_ref[...], h_ref[...]], axis=1)
        f8 = _featurize(xf.reshape((TP + HALO) // 128, 128))
        feat = jnp.concatenate(
            [f.reshape(1, TP + HALO) for f in f8], axis=0)
        a = jnp.concatenate([feat[:, o:o + TP] for o in offs], axis=0)
        o_ref[...] = jnp.maximum(
            jnp.dot(w_ref[...], a, preferred_element_type=jnp.float32), 0.0)
    return body


def _kan_conv_relu(xflat, w81, offs):
    P = xflat.shape[1]
    n = pl.cdiv(P, TP)
    ppad = n * TP
    xbig = jnp.pad(xflat, ((0, 0), (0, ppad + TP - P)))
    out = pl.pallas_call(
        _make_conv_body(offs),
        out_shape=jax.ShapeDtypeStruct((N_CONVS, ppad), jnp.float32),
        grid=(n,),
        in_specs=[
            pl.BlockSpec((1, TP), lambda i: (0, i)),
            pl.BlockSpec((1, HALO), lambda i: (0, (i + 1) * (TP // HALO))),
            pl.BlockSpec((N_CONVS, N_GROUPS * KK), lambda i: (0, 0)),
        ],
        out_specs=pl.BlockSpec((N_CONVS, TP), lambda i: (0, i)),
        compiler_params=pltpu.CompilerParams(dimension_semantics=("parallel",)),
    )(xbig, xbig, w81)
    return out[:, :P]


def _pool2(x):
    *lead, h, w = x.shape
    x = x[..., : h // 2 * 2, : w // 2 * 2]
    x = x.reshape(*lead, h // 2, 2, w // 2, 2)
    return x.max(axis=(-3, -1))


def _mlp_body(x_ref, w1_ref, b1_ref, w2_ref, b2_ref, o_ref):
    h = jnp.dot(x_ref[...], w1_ref[...], preferred_element_type=jnp.float32)
    h = jnp.maximum(h + b1_ref[...], 0.0)
    z = jnp.dot(h, w2_ref[...], preferred_element_type=jnp.float32) + b2_ref[...]
    col = jax.lax.broadcasted_iota(jnp.int32, z.shape, 1)
    valid = col < N_CLASSES
    m = jnp.max(jnp.where(valid, z, -jnp.inf), axis=-1, keepdims=True)
    e = jnp.where(valid, jnp.exp(z - m), 0.0)
    lse = jnp.log(jnp.sum(e, axis=-1, keepdims=True)) + m
    o_ref[...] = jnp.where(valid, z - lse, 0.0)


def kernel(x, c1_w, c2_w, fc1_w, fc1_b, fc2_w, fc2_b):
    B = x.shape[0]
    w81_1 = jnp.transpose(c1_w, (1, 2, 0)).reshape(N_CONVS, N_GROUPS * KK)
    w81_2 = jnp.transpose(c2_w, (1, 2, 0)).reshape(N_CONVS, N_GROUPS * KK)

    o1 = _kan_conv_relu(x.reshape(1, B * 28 * 28), w81_1, OFFS28)
    o1 = o1.reshape(N_CONVS, B, 28, 28)[:, :, :26, :26]
    p1 = _pool2(o1)

    o2 = _kan_conv_relu(p1.reshape(1, N_CONVS * B * 13 * 13), w81_2, OFFS13)
    o2 = o2.reshape(N_CONVS, N_CONVS, B, 13, 13)[..., :11, :11]
    p2 = _pool2(o2)

    feats = p2.reshape(25, B, 25).transpose(1, 0, 2).reshape(B, FC1_IN)
    feats = jnp.pad(feats, ((0, 0), (0, FC1_IN_PAD - FC1_IN)))
    w1 = jnp.pad(fc1_w.reshape(FC1_IN, HIDDEN),
                 ((0, FC1_IN_PAD - FC1_IN), (0, 0)))

    bt = 256
    bpad = pl.cdiv(B, bt) * bt
    feats = jnp.pad(feats, ((0, bpad - B), (0, 0)))
    out = pl.pallas_call(
        _mlp_body,
        out_shape=jax.ShapeDtypeStruct((bpad, FC2_PAD), jnp.float32),
        grid=(bpad // bt,),
        in_specs=[
            pl.BlockSpec((bt, FC1_IN_PAD), lambda i: (i, 0)),
            pl.BlockSpec((FC1_IN_PAD, HIDDEN), lambda i: (0, 0)),
            pl.BlockSpec((1, HIDDEN), lambda i: (0, 0)),
            pl.BlockSpec((HIDDEN, FC2_PAD), lambda i: (0, 0)),
            pl.BlockSpec((1, FC2_PAD), lambda i: (0, 0)),
        ],
        out_specs=pl.BlockSpec((bt, FC2_PAD), lambda i: (i, 0)),
        compiler_params=pltpu.CompilerParams(dimension_semantics=("parallel",)),
    )(feats, w1, fc1_b, fc2_w, fc2_b)
    return out[:B, :N_CLASSES]


# speedup vs baseline: 3.1314x; 1.0222x over previous
import jax
import jax.numpy as jnp
from jax.experimental import pallas as pl
from jax.experimental.pallas import tpu as pltpu

GRID_SIZE = 5
SPLINE_ORDER = 3
N_COEF = GRID_SIZE + SPLINE_ORDER
N_KNOTS = GRID_SIZE + 2 * SPLINE_ORDER + 1
H_STEP = 2.0 / GRID_SIZE
KNOTS = tuple(-1.0 + H_STEP * (i - SPLINE_ORDER) for i in range(N_KNOTS))
N_CONVS = 5
N_GROUPS = 1 + N_COEF
KK = 9

TP = 8192
HALO = 128
HIDDEN = 256
N_CLASSES = 3
FC2_PAD = 128
FC1_IN = 625
FC1_IN_PAD = 640

OFFS28 = tuple(di * 28 + dj for di in range(3) for dj in range(3))
OFFS13 = tuple(di * 13 + dj for di in range(3) for dj in range(3))


def _featurize(x):
    g = KNOTS
    c = [(x >= g[j]).astype(jnp.float32) for j in range(N_KNOTS)]
    bases = [c[j] - c[j + 1] for j in range(N_KNOTS - 1)]
    for k in range(1, SPLINE_ORDER + 1):
        inv_k = 1.0 / (k * H_STEP)
        xk = x * inv_k
        new_bases = []
        t_prev = xk - g[0] * inv_k
        for j in range(len(bases) - 1):
            t_next = xk - g[j + 1] * inv_k
            new_bases.append(t_prev * bases[j] + (1.0 - t_next) * bases[j + 1])
            t_prev = t_next
        bases = new_bases
    silu = x * jax.nn.sigmoid(x)
    return [silu] + bases


def _make_conv_body(offs):
    def body(x_ref, h_ref, w_ref, o_ref):
        xf = jnp.concatenate([x_ref[...], h_ref[...]], axis=1)
        f8 = _featurize(xf.reshape((TP + HALO) // 128, 128))
        feat = jnp.concatenate(
            [f.reshape(1, TP + HALO) for f in f8], axis=0)
        a = jnp.concatenate([feat[:, o:o + TP] for o in offs], axis=0)
        o_ref[...] = jnp.maximum(
            jnp.dot(w_ref[...], a, preferred_element_type=jnp.float32), 0.0)
    return body


def _kan_conv_relu(xflat, w81, offs):
    P = xflat.shape[1]
    n = pl.cdiv(P, TP)
    ppad = n * TP
    xbig = jnp.pad(xflat, ((0, 0), (0, ppad + TP - P)))
    out = pl.pallas_call(
        _make_conv_body(offs),
        out_shape=jax.ShapeDtypeStruct((N_CONVS, ppad), jnp.float32),
        grid=(n,),
        in_specs=[
            pl.BlockSpec((1, TP), lambda i: (0, i)),
            pl.BlockSpec((1, HALO), lambda i: (0, (i + 1) * (TP // HALO))),
            pl.BlockSpec((N_CONVS, N_GROUPS * KK), lambda i: (0, 0)),
        ],
        out_specs=pl.BlockSpec((N_CONVS, TP), lambda i: (0, i)),
        compiler_params=pltpu.CompilerParams(dimension_semantics=("parallel",)),
    )(xbig, xbig, w81)
    return out[:, :P]


def _pool2(x):
    *lead, h, w = x.shape
    x = x[..., : h // 2 * 2, : w // 2 * 2]
    x = x.reshape(*lead, h // 2, 2, w // 2, 2)
    return x.max(axis=(-3, -1))


def _mlp_body(x_ref, w1_ref, b1_ref, w2_ref, b2_ref, o_ref):
    h = jnp.dot(x_ref[...], w1_ref[...], preferred_element_type=jnp.float32)
    h = jnp.maximum(h + b1_ref[...], 0.0)
    z = jnp.dot(h, w2_ref[...], preferred_element_type=jnp.float32) + b2_ref[...]
    col = jax.lax.broadcasted_iota(jnp.int32, z.shape, 1)
    valid = col < N_CLASSES
    m = jnp.max(jnp.where(valid, z, -jnp.inf), axis=-1, keepdims=True)
    e = jnp.where(valid, jnp.exp(z - m), 0.0)
    lse = jnp.log(jnp.sum(e, axis=-1, keepdims=True)) + m
    o_ref[...] = jnp.where(valid, z - lse, 0.0)


def kernel(x, c1_w, c2_w, fc1_w, fc1_b, fc2_w, fc2_b):
    B = x.shape[0]
    w81_1 = jnp.transpose(c1_w, (1, 2, 0)).reshape(N_CONVS, N_GROUPS * KK)
    w81_2 = jnp.transpose(c2_w, (1, 2, 0)).reshape(N_CONVS, N_GROUPS * KK)

    o1 = _kan_conv_relu(x.reshape(1, B * 28 * 28), w81_1, OFFS28)
    o1 = o1.reshape(N_CONVS, B, 28, 28)[:, :, :26, :26]
    p1 = _pool2(o1)

    o2 = _kan_conv_relu(p1.reshape(1, N_CONVS * B * 13 * 13), w81_2, OFFS13)
    o2 = o2.reshape(N_CONVS, N_CONVS, B, 13, 13)[..., :11, :11]
    p2 = _pool2(o2)

    feats = p2.reshape(25, B, 25).transpose(1, 0, 2).reshape(B, FC1_IN)
    feats = jnp.pad(feats, ((0, 0), (0, FC1_IN_PAD - FC1_IN)))
    w1 = jnp.pad(fc1_w.reshape(FC1_IN, HIDDEN),
                 ((0, FC1_IN_PAD - FC1_IN), (0, 0)))

    bt = 256
    bpad = pl.cdiv(B, bt) * bt
    feats = jnp.pad(feats, ((0, bpad - B), (0, 0)))
    out = pl.pallas_call(
        _mlp_body,
        out_shape=jax.ShapeDtypeStruct((bpad, FC2_PAD), jnp.float32),
        grid=(bpad // bt,),
        in_specs=[
            pl.BlockSpec((bt, FC1_IN_PAD), lambda i: (i, 0)),
            pl.BlockSpec((FC1_IN_PAD, HIDDEN), lambda i: (0, 0)),
            pl.BlockSpec((1, HIDDEN), lambda i: (0, 0)),
            pl.BlockSpec((HIDDEN, FC2_PAD), lambda i: (0, 0)),
            pl.BlockSpec((1, FC2_PAD), lambda i: (0, 0)),
        ],
        out_specs=pl.BlockSpec((bt, FC2_PAD), lambda i: (i, 0)),
        compiler_params=pltpu.CompilerParams(dimension_semantics=("parallel",)),
    )(feats, w1, fc1_b, fc2_w, fc2_b)
    return out[:B, :N_CLASSES]
